# baseline (speedup 1.0000x reference)
import functools

import jax
import jax.numpy as jnp
from jax.experimental import pallas as pl
from jax.experimental.pallas import tpu as pltpu

TOP_K = 8
NUM_EXPERTS = 64
BLK = 1024


def _router_body(x_ref, w_ref, b_ref, idx_ref, wgt_ref):
    x = x_ref[...]
    w = w_ref[...]
    logits = jax.lax.dot_general(
        x, w, (((1,), (0,)), ((), ())),
        preferred_element_type=jnp.float32,
    ) + b_ref[...]

    lanes_f = jax.lax.broadcasted_iota(jnp.int32, logits.shape, 1).astype(jnp.float32)
    neg_inf = jnp.float32(-jnp.inf)
    big = jnp.float32(NUM_EXPERTS)
    cur = logits
    vals, idxs = [], []
    for _ in range(TOP_K):
        m = jnp.max(cur, axis=1, keepdims=True)
        idx = jnp.min(jnp.where(cur == m, lanes_f, big),
                      axis=1, keepdims=True)
        vals.append(m)
        idxs.append(idx)
        cur = jnp.where(lanes_f == idx, neg_inf, cur)

    v = jnp.concatenate(vals, axis=1)
    e = jnp.exp(v - v[:, :1])
    wgt_ref[...] = e / jnp.sum(e, axis=1, keepdims=True)
    idx_ref[...] = jnp.concatenate(idxs, axis=1).astype(jnp.int32)


@functools.partial(jax.jit, static_argnames=())
def kernel(x, gate_weight, expert_bias):
    batch, seq, dim = x.shape
    n = batch * seq
    x_flat = x.reshape(n, dim)
    w_t = gate_weight.T
    bias = expert_bias.reshape(1, NUM_EXPERTS)

    grid = (n // BLK,)
    idx, wgt = pl.pallas_call(
        _router_body,
        grid=grid,
        in_specs=[
            pl.BlockSpec((BLK, dim), lambda i: (i, 0)),
            pl.BlockSpec((dim, NUM_EXPERTS), lambda i: (0, 0)),
            pl.BlockSpec((1, NUM_EXPERTS), lambda i: (0, 0)),
        ],
        out_specs=[
            pl.BlockSpec((BLK, TOP_K), lambda i: (i, 0)),
            pl.BlockSpec((BLK, TOP_K), lambda i: (i, 0)),
        ],
        out_shape=[
            jax.ShapeDtypeStruct((n, TOP_K), jnp.int32),
            jax.ShapeDtypeStruct((n, TOP_K), jnp.float32),
        ],
        compiler_params=pltpu.CompilerParams(
            dimension_semantics=("arbitrary",),
        ),
    )(x_flat, w_t, bias)
    return idx, wgt


# speedup vs baseline: 1.3037x; 1.3037x over previous
import functools

import jax
import jax.numpy as jnp
from jax.experimental import pallas as pl
from jax.experimental.pallas import tpu as pltpu

TOP_K = 8
NUM_EXPERTS = 64
BLK = 1024


def _router_body(x_ref, w_ref, b_ref, idx_ref, wgt_ref):
    x = x_ref[...]
    w = w_ref[...]
    logits = jax.lax.dot_general(
        x, w, (((1,), (0,)), ((), ())),
        preferred_element_type=jnp.float32,
    ) + b_ref[...]

    lanes = jax.lax.broadcasted_iota(jnp.int32, logits.shape, 1)
    neg_inf = jnp.float32(-jnp.inf)
    cur = logits
    vals, idxs = [], []
    for _ in range(TOP_K):
        m = jnp.max(cur, axis=1, keepdims=True)
        idx = jnp.argmax(cur, axis=1).reshape(-1, 1)
        vals.append(m)
        idxs.append(idx)
        cur = jnp.where(lanes == idx, neg_inf, cur)

    v = jnp.concatenate(vals, axis=1)
    e = jnp.exp(v - v[:, :1])
    wgt_ref[...] = e / jnp.sum(e, axis=1, keepdims=True)
    idx_ref[...] = jnp.concatenate(idxs, axis=1)


@functools.partial(jax.jit, static_argnames=())
def kernel(x, gate_weight, expert_bias):
    batch, seq, dim = x.shape
    n = batch * seq
    x_flat = x.reshape(n, dim)
    w_t = gate_weight.T
    bias = expert_bias.reshape(1, NUM_EXPERTS)

    grid = (n // BLK,)
    idx, wgt = pl.pallas_call(
        _router_body,
        grid=grid,
        in_specs=[
            pl.BlockSpec((BLK, dim), lambda i: (i, 0)),
            pl.BlockSpec((dim, NUM_EXPERTS), lambda i: (0, 0)),
            pl.BlockSpec((1, NUM_EXPERTS), lambda i: (0, 0)),
        ],
        out_specs=[
            pl.BlockSpec((BLK, TOP_K), lambda i: (i, 0)),
            pl.BlockSpec((BLK, TOP_K), lambda i: (i, 0)),
        ],
        out_shape=[
            jax.ShapeDtypeStruct((n, TOP_K), jnp.int32),
            jax.ShapeDtypeStruct((n, TOP_K), jnp.float32),
        ],
        compiler_params=pltpu.CompilerParams(
            dimension_semantics=("arbitrary",),
        ),
    )(x_flat, w_t, bias)
    return idx, wgt


# speedup vs baseline: 1.3205x; 1.0129x over previous
import functools

import jax
import jax.numpy as jnp
from jax.experimental import pallas as pl
from jax.experimental.pallas import tpu as pltpu

TOP_K = 8
NUM_EXPERTS = 64
BLK = 2048


def _router_body(x_ref, w_ref, b_ref, idx_ref, wgt_ref):
    x = x_ref[...]
    w = w_ref[...]
    logits = jax.lax.dot_general(
        x, w, (((1,), (0,)), ((), ())),
        preferred_element_type=jnp.float32,
    ) + b_ref[...]

    lanes = jax.lax.broadcasted_iota(jnp.int32, logits.shape, 1)
    neg_inf = jnp.float32(-jnp.inf)
    cur = logits
    vals, idxs = [], []
    for _ in range(TOP_K):
        m = jnp.max(cur, axis=1, keepdims=True)
        idx = jnp.argmax(cur, axis=1).reshape(-1, 1)
        vals.append(m)
        idxs.append(idx)
        cur = jnp.where(lanes == idx, neg_inf, cur)

    v = jnp.concatenate(vals, axis=1)
    e = jnp.exp(v - v[:, :1])
    wgt_ref[...] = e / jnp.sum(e, axis=1, keepdims=True)
    idx_ref[...] = jnp.concatenate(idxs, axis=1)


@functools.partial(jax.jit, static_argnames=())
def kernel(x, gate_weight, expert_bias):
    batch, seq, dim = x.shape
    n = batch * seq
    x_flat = x.reshape(n, dim)
    w_t = gate_weight.T
    bias = expert_bias.reshape(1, NUM_EXPERTS)

    grid = (n // BLK,)
    idx, wgt = pl.pallas_call(
        _router_body,
        grid=grid,
        in_specs=[
            pl.BlockSpec((BLK, dim), lambda i: (i, 0)),
            pl.BlockSpec((dim, NUM_EXPERTS), lambda i: (0, 0)),
            pl.BlockSpec((1, NUM_EXPERTS), lambda i: (0, 0)),
        ],
        out_specs=[
            pl.BlockSpec((BLK, TOP_K), lambda i: (i, 0)),
            pl.BlockSpec((BLK, TOP_K), lambda i: (i, 0)),
        ],
        out_shape=[
            jax.ShapeDtypeStruct((n, TOP_K), jnp.int32),
            jax.ShapeDtypeStruct((n, TOP_K), jnp.float32),
        ],
        compiler_params=pltpu.CompilerParams(
            dimension_semantics=("arbitrary",),
        ),
    )(x_flat, w_t, bias)
    return idx, wgt
